# Initial kernel scaffold; baseline (speedup 1.0000x reference)
import functools

import jax
import jax.numpy as jnp
from jax import lax
from jax.experimental import pallas as pl
from jax.experimental.pallas import tpu as pltpu
from jax.experimental.pallas import tpu_sc as plsc

_NC = 2
_NS = 16
_NW = _NC * _NS
_B = 16384
_D = 512
_BW = _B // _NW
_C = 64
_NCHUNK = _BW // _C
_L = 16


def _dot_chunk(rows_v, w_v, w_off, out_v, out_base, first):
  lane = lax.iota(jnp.int32, _L)

  def g_body(g, carry):
    out_vec = jnp.zeros((_L,), jnp.float32)
    for rr in range(_L):
      accs = [jnp.zeros((_L,), jnp.float32) for _ in range(4)]
      for j in range(_D // _L):
        accs[j % 4] = accs[j % 4] + (
            rows_v[g * _L + rr, pl.ds(j * _L, _L)]
            * w_v[pl.ds(w_off + j * _L, _L)]
        )
      acc = (accs[0] + accs[1]) + (accs[2] + accs[3])
      tot = jnp.sum(acc)
      out_vec = jnp.where(lane == rr, tot, out_vec)
    idx = pl.multiple_of(out_base + g * _L, _L)
    if first:
      out_v[pl.ds(idx, _L)] = out_vec
    else:
      out_v[pl.ds(idx, _L)] = out_v[pl.ds(idx, _L)] + out_vec
    return carry

  lax.fori_loop(0, _C // _L, g_body, 0)


def _make_sc_kernel():
  mesh = plsc.VectorSubcoreMesh(core_axis_name="c", subcore_axis_name="s")

  @functools.partial(
      pl.kernel,
      mesh=mesh,
      out_type=jax.ShapeDtypeStruct((_B,), jnp.float32),
      scratch_types=[
          pltpu.VMEM((_D * 2,), jnp.float32),
          pltpu.VMEM((_C,), jnp.int32),
          pltpu.VMEM((_C,), jnp.int32),
          pltpu.VMEM((_C, _D), jnp.float32),
          pltpu.VMEM((_C, _D), jnp.float32),
          pltpu.VMEM((_BW,), jnp.float32),
          pltpu.SemaphoreType.DMA,
          pltpu.SemaphoreType.DMA,
      ],
  )
  def sc_kernel(table_hbm, idx0_hbm, idx1_hbm, w_hbm, out_hbm,
                w_v, ib0, ib1, rb0, rb1, out_v, sem0, sem1):
    wid = lax.axis_index("s") * _NC + lax.axis_index("c")
    base = pl.multiple_of(wid * _BW, _BW)
    pltpu.sync_copy(w_hbm, w_v)

    def chunk_body(c, carry):
      off = pl.multiple_of(base + c * _C, _C)
      pltpu.sync_copy(idx0_hbm.at[pl.ds(off, _C)], ib0)
      pltpu.sync_copy(idx1_hbm.at[pl.ds(off, _C)], ib1)
      cp0 = pltpu.async_copy(table_hbm.at[ib0], rb0, sem0)
      cp1 = pltpu.async_copy(table_hbm.at[ib1], rb1, sem1)
      cp0.wait()
      _dot_chunk(rb0, w_v, 0, out_v, c * _C, True)
      cp1.wait()
      _dot_chunk(rb1, w_v, _D, out_v, c * _C, False)
      return carry

    lax.fori_loop(0, _NCHUNK, chunk_body, 0)
    pltpu.sync_copy(out_v, out_hbm.at[pl.ds(base, _BW)])

  return sc_kernel


_sc_kernel = _make_sc_kernel()


@jax.jit
def kernel(x, table, W, b):
  idx0 = x[:, 0].astype(jnp.int32)
  idx1 = x[:, 1].astype(jnp.int32)
  w = W.reshape(-1).astype(jnp.float32)
  out = _sc_kernel(table, idx0, idx1, w)
  return out.reshape(_B, 1) + b


# speedup vs baseline: 7.2061x; 7.2061x over previous
import functools

import jax
import jax.numpy as jnp
from jax import lax
from jax.experimental import pallas as pl
from jax.experimental.pallas import tpu as pltpu
from jax.experimental.pallas import tpu_sc as plsc

_NC = 2
_NS = 16
_NW = _NC * _NS
_B = 16384
_D = 512
_BW = _B // _NW
_C = 64
_NCHUNK = _BW // _C
_L = 16


def _lane_shuffle(v, idx):
  dnums = lax.GatherDimensionNumbers(
      offset_dims=(), collapsed_slice_dims=(0,), start_index_map=(0,))
  return lax.gather(v, idx[:, None], dnums, slice_sizes=(1,),
                    mode=lax.GatherScatterMode.PROMISE_IN_BOUNDS)


def _hsum16(vecs, lane):
  s = 1
  while len(vecs) > 1:
    nxt = []
    for k in range(0, len(vecs), 2):
      u, w = vecs[k], vecs[k + 1]
      m = (lane & s) == 0
      a = jnp.where(m, u, w)
      b = jnp.where(m, w, u)
      nxt.append(a + _lane_shuffle(b, lane ^ s))
    vecs = nxt
    s *= 2
  return vecs[0]


def _dot_chunk(rows_v, w_v, w_off, out_v, out_base, first):
  lane = lax.iota(jnp.int32, _L)

  def g_body(g, carry):
    accs = [jnp.zeros((_L,), jnp.float32) for _ in range(_L)]
    for j in range(_D // _L):
      wv = w_v[pl.ds(w_off + j * _L, _L)]
      for rr in range(_L):
        accs[rr] = accs[rr] + rows_v[g * _L + rr, pl.ds(j * _L, _L)] * wv
    out_vec = _hsum16(accs, lane)
    idx = pl.multiple_of(out_base + g * _L, _L)
    if first:
      out_v[pl.ds(idx, _L)] = out_vec
    else:
      out_v[pl.ds(idx, _L)] = out_v[pl.ds(idx, _L)] + out_vec
    return carry

  lax.fori_loop(0, _C // _L, g_body, 0)


def _make_sc_kernel():
  mesh = plsc.VectorSubcoreMesh(core_axis_name="c", subcore_axis_name="s")

  @functools.partial(
      pl.kernel,
      mesh=mesh,
      out_type=jax.ShapeDtypeStruct((_B,), jnp.float32),
      scratch_types=[
          pltpu.VMEM((_D * 2,), jnp.float32),
          pltpu.VMEM((_C,), jnp.int32),
          pltpu.VMEM((_C,), jnp.int32),
          pltpu.VMEM((_C, _D), jnp.float32),
          pltpu.VMEM((_C, _D), jnp.float32),
          pltpu.VMEM((_BW,), jnp.float32),
          pltpu.SemaphoreType.DMA,
          pltpu.SemaphoreType.DMA,
      ],
  )
  def sc_kernel(table_hbm, idx0_hbm, idx1_hbm, w_hbm, out_hbm,
                w_v, ib0, ib1, rb0, rb1, out_v, sem0, sem1):
    wid = lax.axis_index("s") * _NC + lax.axis_index("c")
    base = pl.multiple_of(wid * _BW, _BW)
    pltpu.sync_copy(w_hbm, w_v)

    def chunk_body(c, carry):
      off = pl.multiple_of(base + c * _C, _C)
      pltpu.sync_copy(idx0_hbm.at[pl.ds(off, _C)], ib0)
      pltpu.sync_copy(idx1_hbm.at[pl.ds(off, _C)], ib1)
      cp0 = pltpu.async_copy(table_hbm.at[ib0], rb0, sem0)
      cp1 = pltpu.async_copy(table_hbm.at[ib1], rb1, sem1)
      cp0.wait()
      _dot_chunk(rb0, w_v, 0, out_v, c * _C, True)
      cp1.wait()
      _dot_chunk(rb1, w_v, _D, out_v, c * _C, False)
      return carry

    lax.fori_loop(0, _NCHUNK, chunk_body, 0)
    pltpu.sync_copy(out_v, out_hbm.at[pl.ds(base, _BW)])

  return sc_kernel


_sc_kernel = _make_sc_kernel()


@jax.jit
def kernel(x, table, W, b):
  idx0 = x[:, 0].astype(jnp.int32)
  idx1 = x[:, 1].astype(jnp.int32)
  w = W.reshape(-1).astype(jnp.float32)
  out = _sc_kernel(table, idx0, idx1, w)
  return out.reshape(_B, 1) + b


# speedup vs baseline: 7.7279x; 1.0724x over previous
import functools

import jax
import jax.numpy as jnp
from jax import lax
from jax.experimental import pallas as pl
from jax.experimental.pallas import tpu as pltpu
from jax.experimental.pallas import tpu_sc as plsc

_NC = 2
_NS = 16
_NW = _NC * _NS
_B = 16384
_D = 512
_BW = _B // _NW
_C = 32
_NCHUNK = _BW // _C
_L = 16


def _lane_shuffle(v, idx):
  dnums = lax.GatherDimensionNumbers(
      offset_dims=(), collapsed_slice_dims=(0,), start_index_map=(0,))
  return lax.gather(v, idx[:, None], dnums, slice_sizes=(1,),
                    mode=lax.GatherScatterMode.PROMISE_IN_BOUNDS)


def _hsum16(vecs, lane):
  s = 1
  while len(vecs) > 1:
    nxt = []
    for k in range(0, len(vecs), 2):
      u, w = vecs[k], vecs[k + 1]
      m = (lane & s) == 0
      a = jnp.where(m, u, w)
      b = jnp.where(m, w, u)
      nxt.append(a + _lane_shuffle(b, lane ^ s))
    vecs = nxt
    s *= 2
  return vecs[0]


def _dot_chunk(rows_v, w_v, w_off, out_v, out_base, first):
  lane = lax.iota(jnp.int32, _L)

  def g_body(g, carry):
    accs = [jnp.zeros((_L,), jnp.float32) for _ in range(_L)]
    for j in range(_D // _L):
      wv = w_v[pl.ds(w_off + j * _L, _L)]
      for rr in range(_L):
        accs[rr] = accs[rr] + rows_v[g * _L + rr, pl.ds(j * _L, _L)] * wv
    out_vec = _hsum16(accs, lane)
    idx = pl.multiple_of(out_base + g * _L, _L)
    if first:
      out_v[pl.ds(idx, _L)] = out_vec
    else:
      out_v[pl.ds(idx, _L)] = out_v[pl.ds(idx, _L)] + out_vec
    return carry

  lax.fori_loop(0, _C // _L, g_body, 0)


def _make_sc_kernel():
  mesh = plsc.VectorSubcoreMesh(core_axis_name="c", subcore_axis_name="s")

  @functools.partial(
      pl.kernel,
      mesh=mesh,
      out_type=jax.ShapeDtypeStruct((_B,), jnp.float32),
      scratch_types=[
          pltpu.VMEM((_D * 2,), jnp.float32),
          pltpu.VMEM((_BW,), jnp.int32),
          pltpu.VMEM((_BW,), jnp.int32),
          pltpu.VMEM((_C, _D), jnp.float32),
          pltpu.VMEM((_C, _D), jnp.float32),
          pltpu.VMEM((_C, _D), jnp.float32),
          pltpu.VMEM((_C, _D), jnp.float32),
          pltpu.VMEM((_BW,), jnp.float32),
          pltpu.SemaphoreType.DMA,
          pltpu.SemaphoreType.DMA,
          pltpu.SemaphoreType.DMA,
          pltpu.SemaphoreType.DMA,
      ],
  )
  def sc_kernel(table_hbm, idx0_hbm, idx1_hbm, w_hbm, out_hbm,
                w_v, idx0_v, idx1_v, rbA0, rbA1, rbB0, rbB1, out_v,
                semA0, semA1, semB0, semB1):
    wid = lax.axis_index("s") * _NC + lax.axis_index("c")
    base = pl.multiple_of(wid * _BW, _BW)
    pltpu.sync_copy(w_hbm, w_v)
    pltpu.sync_copy(idx0_hbm.at[pl.ds(base, _BW)], idx0_v)
    pltpu.sync_copy(idx1_hbm.at[pl.ds(base, _BW)], idx1_v)

    def gathers(c, rb0, rb1, sem0, sem1):
      off = pl.multiple_of(c * _C, _C)
      cp0 = pltpu.make_async_copy(
          table_hbm.at[idx0_v.at[pl.ds(off, _C)]], rb0, sem0)
      cp1 = pltpu.make_async_copy(
          table_hbm.at[idx1_v.at[pl.ds(off, _C)]], rb1, sem1)
      return cp0, cp1

    def start(c, rb0, rb1, sem0, sem1):
      cp0, cp1 = gathers(c, rb0, rb1, sem0, sem1)
      cp0.start()
      cp1.start()

    def wait_compute(c, rb0, rb1, sem0, sem1):
      cp0, cp1 = gathers(c, rb0, rb1, sem0, sem1)
      cp0.wait()
      _dot_chunk(rb0, w_v, 0, out_v, c * _C, True)
      cp1.wait()
      _dot_chunk(rb1, w_v, _D, out_v, c * _C, False)

    start(0, rbA0, rbA1, semA0, semA1)
    start(1, rbB0, rbB1, semB0, semB1)

    def pair_body(p, carry):
      cA = 2 * p
      wait_compute(cA, rbA0, rbA1, semA0, semA1)

      @pl.when(p < _NCHUNK // 2 - 1)
      def _():
        start(cA + 2, rbA0, rbA1, semA0, semA1)

      wait_compute(cA + 1, rbB0, rbB1, semB0, semB1)

      @pl.when(p < _NCHUNK // 2 - 1)
      def _():
        start(cA + 3, rbB0, rbB1, semB0, semB1)

      return carry

    lax.fori_loop(0, _NCHUNK // 2, pair_body, 0)
    pltpu.sync_copy(out_v, out_hbm.at[pl.ds(base, _BW)])

  return sc_kernel


_sc_kernel = _make_sc_kernel()


@jax.jit
def kernel(x, table, W, b):
  idx0 = x[:, 0].astype(jnp.int32)
  idx1 = x[:, 1].astype(jnp.int32)
  w = W.reshape(-1).astype(jnp.float32)
  out = _sc_kernel(table, idx0, idx1, w)
  return out.reshape(_B, 1) + b


# speedup vs baseline: 14.7453x; 1.9081x over previous
import functools

import jax
import jax.numpy as jnp
from jax import lax
from jax.experimental import pallas as pl
from jax.experimental.pallas import tpu as pltpu
from jax.experimental.pallas import tpu_sc as plsc

_NC = 2
_NS = 16
_NW = _NC * _NS
_B = 16384
_D = 512
_BW = _B // _NW
_C = 32
_NCHUNK = _BW // _C
_L = 16


def _lane_shuffle(v, idx):
  dnums = lax.GatherDimensionNumbers(
      offset_dims=(), collapsed_slice_dims=(0,), start_index_map=(0,))
  return lax.gather(v, idx[:, None], dnums, slice_sizes=(1,),
                    mode=lax.GatherScatterMode.PROMISE_IN_BOUNDS)


def _hsum16(vecs, lane):
  s = 1
  while len(vecs) > 1:
    nxt = []
    for k in range(0, len(vecs), 2):
      u, w = vecs[k], vecs[k + 1]
      m = (lane & s) == 0
      a = jnp.where(m, u, w)
      b = jnp.where(m, w, u)
      nxt.append(a + _lane_shuffle(b, lane ^ s))
    vecs = nxt
    s *= 2
  return vecs[0]


def _dot_chunk(rows_v, w_v, w_off, out_v, out_base, first):
  lane = lax.iota(jnp.int32, _L)
  zero = jnp.zeros((_L,), jnp.float32)

  def g_body(g, carry):
    @plsc.parallel_loop(0, _D // _L, unroll=4,
                        carry=tuple(zero for _ in range(_L)))
    def accs(j, acc):
      joff = pl.multiple_of(j * _L, _L)
      wv = w_v[pl.ds(pl.multiple_of(w_off + joff, _L), _L)]
      return tuple(
          acc[rr] + rows_v[g * _L + rr, pl.ds(joff, _L)] * wv
          for rr in range(_L))

    out_vec = _hsum16(list(accs), lane)
    idx = pl.multiple_of(out_base + g * _L, _L)
    if first:
      out_v[pl.ds(idx, _L)] = out_vec
    else:
      out_v[pl.ds(idx, _L)] = out_v[pl.ds(idx, _L)] + out_vec
    return carry

  lax.fori_loop(0, _C // _L, g_body, 0)


def _make_sc_kernel():
  mesh = plsc.VectorSubcoreMesh(core_axis_name="c", subcore_axis_name="s")

  @functools.partial(
      pl.kernel,
      mesh=mesh,
      out_type=jax.ShapeDtypeStruct((_B,), jnp.float32),
      scratch_types=[
          pltpu.VMEM((_D * 2,), jnp.float32),
          pltpu.VMEM((_BW,), jnp.int32),
          pltpu.VMEM((_BW,), jnp.int32),
          pltpu.VMEM((_C, _D), jnp.float32),
          pltpu.VMEM((_C, _D), jnp.float32),
          pltpu.VMEM((_C, _D), jnp.float32),
          pltpu.VMEM((_C, _D), jnp.float32),
          pltpu.VMEM((_BW,), jnp.float32),
          pltpu.SemaphoreType.DMA,
          pltpu.SemaphoreType.DMA,
          pltpu.SemaphoreType.DMA,
          pltpu.SemaphoreType.DMA,
      ],
  )
  def sc_kernel(table_hbm, idx0_hbm, idx1_hbm, w_hbm, out_hbm,
                w_v, idx0_v, idx1_v, rbA0, rbA1, rbB0, rbB1, out_v,
                semA0, semA1, semB0, semB1):
    wid = lax.axis_index("s") * _NC + lax.axis_index("c")
    base = pl.multiple_of(wid * _BW, _BW)
    pltpu.sync_copy(w_hbm, w_v)
    pltpu.sync_copy(idx0_hbm.at[pl.ds(base, _BW)], idx0_v)
    pltpu.sync_copy(idx1_hbm.at[pl.ds(base, _BW)], idx1_v)

    def gathers(c, rb0, rb1, sem0, sem1):
      off = pl.multiple_of(c * _C, _C)
      cp0 = pltpu.make_async_copy(
          table_hbm.at[idx0_v.at[pl.ds(off, _C)]], rb0, sem0)
      cp1 = pltpu.make_async_copy(
          table_hbm.at[idx1_v.at[pl.ds(off, _C)]], rb1, sem1)
      return cp0, cp1

    def start(c, rb0, rb1, sem0, sem1):
      cp0, cp1 = gathers(c, rb0, rb1, sem0, sem1)
      cp0.start()
      cp1.start()

    def wait_compute(c, rb0, rb1, sem0, sem1):
      cp0, cp1 = gathers(c, rb0, rb1, sem0, sem1)
      cp0.wait()
      _dot_chunk(rb0, w_v, 0, out_v, c * _C, True)
      cp1.wait()
      _dot_chunk(rb1, w_v, _D, out_v, c * _C, False)

    start(0, rbA0, rbA1, semA0, semA1)
    start(1, rbB0, rbB1, semB0, semB1)

    def pair_body(p, carry):
      cA = 2 * p
      wait_compute(cA, rbA0, rbA1, semA0, semA1)

      @pl.when(p < _NCHUNK // 2 - 1)
      def _():
        start(cA + 2, rbA0, rbA1, semA0, semA1)

      wait_compute(cA + 1, rbB0, rbB1, semB0, semB1)

      @pl.when(p < _NCHUNK // 2 - 1)
      def _():
        start(cA + 3, rbB0, rbB1, semB0, semB1)

      return carry

    lax.fori_loop(0, _NCHUNK // 2, pair_body, 0)
    pltpu.sync_copy(out_v, out_hbm.at[pl.ds(base, _BW)])

  return sc_kernel


_sc_kernel = _make_sc_kernel()


@jax.jit
def kernel(x, table, W, b):
  idx0 = x[:, 0].astype(jnp.int32)
  idx1 = x[:, 1].astype(jnp.int32)
  w = W.reshape(-1).astype(jnp.float32)
  out = _sc_kernel(table, idx0, idx1, w)
  return out.reshape(_B, 1) + b
